# baseline (speedup 1.0000x reference)
import functools

import jax
import jax.numpy as jnp
from jax.experimental import pallas as pl
from jax.experimental.pallas import tpu as pltpu

_KS = 7
_PD = 3


def _conv7x7(p, w_ref, w_base, th, tw, height, width):
    S = height * width
    cols = [None] * _KS
    for k in range(_KS):
        dh = k - _PD
        ph = pltpu.roll(p, (-dh * width) % S, 1)
        if dh:
            ph = jnp.where((th + dh >= 0) & (th + dh < height), ph, 0.0)
        for l in range(_KS):
            t = ph * w_ref[w_base + k * _KS + l]
            cols[l] = t if cols[l] is None else cols[l] + t
    y = None
    for l in range(_KS):
        dw = l - _PD
        yl = pltpu.roll(cols[l], (-dw) % S, 1)
        if dw:
            yl = jnp.where((tw + dw >= 0) & (tw + dw < width), yl, 0.0)
        y = yl if y is None else y + yl
    return y


def _body(x_ref, w_ref, o_ref, *, sub, cmul, height, width):
    Bt, C, S = x_ref.shape
    f32 = jnp.float32
    lane = jax.lax.broadcasted_iota(jnp.int32, (sub, S), 1)
    th = lane // width
    tw = lane - th * width

    gates = []
    for s0 in range(0, Bt, sub):
        xs = x_ref[s0:s0 + sub]
        mx = jnp.max(xs, axis=1)
        sm = jnp.sum(xs, axis=1) * (1.0 / C)
        y = (_conv7x7(mx, w_ref, 0, th, tw, height, width)
             + _conv7x7(sm, w_ref, _KS * _KS, th, tw, height, width)
             + w_ref[2 * _KS * _KS])
        gates.append(jax.nn.sigmoid(y))
    gate = jnp.concatenate(gates, axis=0) if len(gates) > 1 else gates[0]

    for c0 in range(0, C, cmul):
        o_ref[:, c0:c0 + cmul, :] = x_ref[:, c0:c0 + cmul, :] * gate[:, None, :]


def kernel(x, conv_w, conv_b):
    B, C, H, W = x.shape
    S = H * W

    wb = jnp.concatenate([conv_w.reshape(2 * _KS * _KS),
                          conv_b.reshape(1)]).astype(jnp.float32)

    x_flat = x.reshape(B, C, S)

    bt = 16
    while B % bt:
        bt //= 2
    sub = min(8, bt)
    cmul = 8 if C % 8 == 0 else C

    body = functools.partial(_body, sub=sub, cmul=cmul, height=H, width=W)
    out = pl.pallas_call(
        body,
        out_shape=jax.ShapeDtypeStruct((B, C, S), x.dtype),
        grid=(B // bt,),
        in_specs=[
            pl.BlockSpec((bt, C, S), lambda i: (i, 0, 0)),
            pl.BlockSpec(memory_space=pltpu.MemorySpace.SMEM),
        ],
        out_specs=pl.BlockSpec((bt, C, S), lambda i: (i, 0, 0)),
        compiler_params=pltpu.CompilerParams(
            dimension_semantics=("parallel",),
            vmem_limit_bytes=int(56 << 20),
        ),
    )(x_flat, wb)

    return out.reshape(B, C, H, W)


# speedup vs baseline: 1.0135x; 1.0135x over previous
import functools

import jax
import jax.numpy as jnp
from jax.experimental import pallas as pl
from jax.experimental.pallas import tpu as pltpu

_KS = 7
_PD = 3


def _conv7x7(p, w_ref, w_base, th, tw, height, width):
    S = height * width
    cols = [None] * _KS
    for k in range(_KS):
        dh = k - _PD
        ph = pltpu.roll(p, (-dh * width) % S, 1)
        if dh:
            ph = jnp.where((th + dh >= 0) & (th + dh < height), ph, 0.0)
        for l in range(_KS):
            t = ph * w_ref[w_base + k * _KS + l]
            cols[l] = t if cols[l] is None else cols[l] + t
    y = None
    for l in range(_KS):
        dw = l - _PD
        yl = pltpu.roll(cols[l], (-dw) % S, 1)
        if dw:
            yl = jnp.where((tw + dw >= 0) & (tw + dw < width), yl, 0.0)
        y = yl if y is None else y + yl
    return y


def _body(x_ref, w_ref, o_ref, *, sub, cmul, height, width):
    Bt, C, S = x_ref.shape
    f32 = jnp.float32
    lane = jax.lax.broadcasted_iota(jnp.int32, (sub, S), 1)
    th = lane // width
    tw = lane - th * width

    gates = []
    for s0 in range(0, Bt, sub):
        xs = x_ref[s0:s0 + sub]
        mx = jnp.max(xs, axis=1)
        sm = jnp.sum(xs, axis=1) * (1.0 / C)
        y = (_conv7x7(mx, w_ref, 0, th, tw, height, width)
             + _conv7x7(sm, w_ref, _KS * _KS, th, tw, height, width)
             + w_ref[2 * _KS * _KS])
        gates.append(jax.nn.sigmoid(y))
    gate = jnp.concatenate(gates, axis=0) if len(gates) > 1 else gates[0]

    for c0 in range(0, C, cmul):
        o_ref[:, c0:c0 + cmul, :] = x_ref[:, c0:c0 + cmul, :] * gate[:, None, :]


def kernel(x, conv_w, conv_b):
    B, C, H, W = x.shape
    S = H * W

    wb = jnp.concatenate([conv_w.reshape(2 * _KS * _KS),
                          conv_b.reshape(1)]).astype(jnp.float32)

    x_flat = x.reshape(B, C, S)

    bt = 32
    while B % bt:
        bt //= 2
    sub = min(8, bt)
    cmul = 8 if C % 8 == 0 else C

    body = functools.partial(_body, sub=sub, cmul=cmul, height=H, width=W)
    out = pl.pallas_call(
        body,
        out_shape=jax.ShapeDtypeStruct((B, C, S), x.dtype),
        grid=(B // bt,),
        in_specs=[
            pl.BlockSpec((bt, C, S), lambda i: (i, 0, 0)),
            pl.BlockSpec(memory_space=pltpu.MemorySpace.SMEM),
        ],
        out_specs=pl.BlockSpec((bt, C, S), lambda i: (i, 0, 0)),
        compiler_params=pltpu.CompilerParams(
            dimension_semantics=("parallel",),
            vmem_limit_bytes=int(56 << 20),
        ),
    )(x_flat, wb)

    return out.reshape(B, C, H, W)
